# Initial kernel scaffold; baseline (speedup 1.0000x reference)
import jax
import jax.numpy as jnp
from jax.experimental import pallas as pl


def kernel(contour_tensor, point_tensor, location_tensor, on_curve_tensor, contour_table, point_table, oncurve_table, loc_W, loc_b):
    raise NotImplementedError("write your pallas kernel here")


# speedup vs baseline: 4.8444x; 4.8444x over previous
import functools

import jax
import jax.numpy as jnp
from jax import lax
from jax.experimental import pallas as pl
from jax.experimental.pallas import tpu as pltpu
from jax.experimental.pallas import tpu_sc as plsc

EMBED_DIM = 64
NC = 2
NS = 16
NW = NC * NS
CHUNK = 128


def _sc_body(ct_hbm, pt_hbm, ci_hbm, pi_hbm, oc_hbm, x_hbm, y_hbm, w_hbm,
             out_hbm,
             ci_v, pi_v, oc_v, pi2_v, x_v, y_v, rc_v, rp_v, o_v, w_v,
             sem_c, sem_p, n_tokens):
    per_w = n_tokens // NW
    n_chunks = per_w // CHUNK
    wid = lax.axis_index("s") * NC + lax.axis_index("c")
    base = wid * per_w

    pltpu.sync_copy(w_hbm, w_v)
    w0 = [w_v[0, pl.ds(16 * j, 16)] for j in range(4)]
    w1 = [w_v[1, pl.ds(16 * j, 16)] for j in range(4)]

    def chunk_body(ck, carry):
        off = base + ck * CHUNK
        pltpu.sync_copy(ci_hbm.at[pl.ds(off, CHUNK)], ci_v)
        pltpu.sync_copy(pi_hbm.at[pl.ds(off, CHUNK)], pi_v)
        pltpu.sync_copy(oc_hbm.at[pl.ds(off, CHUNK)], oc_v)
        pltpu.sync_copy(x_hbm.at[pl.ds(off, CHUNK)], x_v)
        pltpu.sync_copy(y_hbm.at[pl.ds(off, CHUNK)], y_v)
        for g in range(CHUNK // 16):
            s = pl.ds(16 * g, 16)
            pi2_v[s] = pi_v[s] * 2 + oc_v[s]
        cp_c = pltpu.async_copy(ct_hbm.at[ci_v], rc_v, sem_c)
        cp_p = pltpu.async_copy(pt_hbm.at[pi2_v], rp_v, sem_p)
        cp_c.wait()
        cp_p.wait()

        def grp_body(g, tc):
            sg = pl.ds(16 * g, 16)
            xg = x_v[sg]
            yg = y_v[sg]
            for i in range(16):
                xs = xg[i]
                ys = yg[i]
                t = 16 * g + i
                for j in range(4):
                    s = pl.ds(16 * j, 16)
                    o_v[t, s] = (rc_v[t, s] + rp_v[t, s]) + (xs * w0[j] + ys * w1[j])
            return tc

        lax.fori_loop(0, CHUNK // 16, grp_body, 0)
        pltpu.sync_copy(o_v, out_hbm.at[pl.ds(off, CHUNK)])
        return carry

    lax.fori_loop(0, n_chunks, chunk_body, 0)


def kernel(contour_tensor, point_tensor, location_tensor, on_curve_tensor,
           contour_table, point_table, oncurve_table, loc_W, loc_b):
    B, L = contour_tensor.shape
    n = B * L
    scale = float(EMBED_DIM) ** 0.5

    ct2 = contour_table[1:] * scale
    pt2 = ((point_table[1:, None, :] + oncurve_table[None, 1:3, :]
            + loc_b[None, None, :]) * scale).reshape(-1, EMBED_DIM)
    w8 = loc_W.T * scale

    ci = contour_tensor.reshape(n).astype(jnp.int32)
    pi = point_tensor.reshape(n).astype(jnp.int32)
    oc = on_curve_tensor.reshape(n).astype(jnp.int32)
    x = location_tensor[..., 0].reshape(n)
    y = location_tensor[..., 1].reshape(n)

    mesh = plsc.VectorSubcoreMesh(core_axis_name="c", subcore_axis_name="s")
    run = functools.partial(
        pl.kernel,
        mesh=mesh,
        compiler_params=pltpu.CompilerParams(use_tc_tiling_on_sc=False),
        out_type=jax.ShapeDtypeStruct((n, EMBED_DIM), jnp.float32),
        scratch_types=[
            pltpu.VMEM((CHUNK,), jnp.int32),
            pltpu.VMEM((CHUNK,), jnp.int32),
            pltpu.VMEM((CHUNK,), jnp.int32),
            pltpu.VMEM((CHUNK,), jnp.int32),
            pltpu.VMEM((CHUNK,), jnp.float32),
            pltpu.VMEM((CHUNK,), jnp.float32),
            pltpu.VMEM((CHUNK, EMBED_DIM), jnp.float32),
            pltpu.VMEM((CHUNK, EMBED_DIM), jnp.float32),
            pltpu.VMEM((CHUNK, EMBED_DIM), jnp.float32),
            pltpu.VMEM((2, EMBED_DIM), jnp.float32),
            pltpu.SemaphoreType.DMA,
            pltpu.SemaphoreType.DMA,
        ],
    )(functools.partial(_sc_body, n_tokens=n))
    out = run(ct2, pt2, ci, pi, oc, x, y, w8)
    return out.reshape(B, L, EMBED_DIM)
